# Initial kernel scaffold; baseline (speedup 1.0000x reference)
import jax
import jax.numpy as jnp
from jax.experimental import pallas as pl


def kernel(ids, translation, log_var):
    raise NotImplementedError("write your pallas kernel here")


# speedup vs baseline: 3.4352x; 3.4352x over previous
import functools
import math

import jax
import jax.numpy as jnp
from jax import lax
from jax.experimental import pallas as pl
from jax.experimental.pallas import tpu as pltpu
from jax.experimental.pallas import tpu_sc as plsc

MIN_VAR = 0.02
MAX_VAR = 3.0

_CHUNK = 128


def _var_table_body(lv_ref, var_ref):
    var_ref[...] = jnp.minimum(jax.nn.softplus(lv_ref[...]) + MIN_VAR, MAX_VAR)


def _make_gather(num_rows, dim, nc, ns):
    nw = nc * ns
    per_w = num_rows // nw
    n_chunks = per_w // _CHUNK
    mesh = plsc.VectorSubcoreMesh(core_axis_name="c", subcore_axis_name="s")
    out_t = jax.ShapeDtypeStruct((num_rows, dim), jnp.float32)

    @functools.partial(
        pl.kernel,
        out_type=(out_t, out_t),
        mesh=mesh,
        scratch_types=[
            pltpu.VMEM((_CHUNK,), jnp.int32),
            pltpu.VMEM((_CHUNK, dim), jnp.float32),
            pltpu.VMEM((_CHUNK, dim), jnp.float32),
            pltpu.SemaphoreType.DMA,
            pltpu.SemaphoreType.DMA,
        ],
    )
    def gather_k(ids_hbm, mu_tab, var_tab, mu_out, var_out,
                 idx_v, mu_v, var_v, sem_mu, sem_var):
        wid = lax.axis_index("s") * nc + lax.axis_index("c")
        base = wid * per_w

        def chunk_body(i, carry):
            off = base + i * _CHUNK
            pltpu.sync_copy(ids_hbm.at[pl.ds(off, _CHUNK)], idx_v)
            g_mu = pltpu.async_copy(mu_tab.at[idx_v], mu_v, sem_mu)
            g_var = pltpu.async_copy(var_tab.at[idx_v], var_v, sem_var)
            g_mu.wait()
            pltpu.sync_copy(mu_v, mu_out.at[pl.ds(off, _CHUNK)])
            g_var.wait()
            pltpu.sync_copy(var_v, var_out.at[pl.ds(off, _CHUNK)])
            return carry

        lax.fori_loop(0, n_chunks, chunk_body, 0)

    return gather_k


def kernel(ids, translation, log_var):
    var_table = pl.pallas_call(
        _var_table_body,
        out_shape=jax.ShapeDtypeStruct(log_var.shape, jnp.float32),
    )(log_var)

    info = plsc.get_sparse_core_info()
    num_rows = ids.size
    dim = translation.shape[1]
    ids_flat = ids.reshape(num_rows)
    gather_k = _make_gather(num_rows, dim, info.num_cores, info.num_subcores)
    mu_flat, var_flat = gather_k(ids_flat, translation, var_table)
    out_shape = ids.shape + (dim,)
    return mu_flat.reshape(out_shape), var_flat.reshape(out_shape)


# speedup vs baseline: 3.5731x; 1.0401x over previous
import functools
import math

import jax
import jax.numpy as jnp
from jax import lax
from jax.experimental import pallas as pl
from jax.experimental.pallas import tpu as pltpu
from jax.experimental.pallas import tpu_sc as plsc

MIN_VAR = 0.02
MAX_VAR = 3.0

_CHUNK = 128


def _var_table_body(lv_ref, var_ref):
    var_ref[...] = jnp.minimum(jax.nn.softplus(lv_ref[...]) + MIN_VAR, MAX_VAR)


def _make_gather(num_rows, dim, nc, ns):
    nw = nc * ns
    per_w = num_rows // nw
    n_chunks = per_w // _CHUNK
    mesh = plsc.VectorSubcoreMesh(core_axis_name="c", subcore_axis_name="s")
    out_t = jax.ShapeDtypeStruct((num_rows, dim), jnp.float32)

    @functools.partial(
        pl.kernel,
        out_type=(out_t, out_t),
        mesh=mesh,
        scratch_types=[
            pltpu.VMEM((per_w,), jnp.int32),
            pltpu.VMEM((2, _CHUNK, dim), jnp.float32),
            pltpu.VMEM((2, _CHUNK, dim), jnp.float32),
            pltpu.SemaphoreType.DMA,
            pltpu.SemaphoreType.DMA,
            pltpu.SemaphoreType.DMA,
            pltpu.SemaphoreType.DMA,
        ],
    )
    def gather_k(ids_hbm, mu_tab, var_tab, mu_out, var_out,
                 idx_all, mu_v, var_v, sg0, sg1, ss0, ss1):
        wid = lax.axis_index("s") * nc + lax.axis_index("c")
        base = wid * per_w
        pltpu.sync_copy(ids_hbm.at[pl.ds(base, per_w)], idx_all)
        sg = (sg0, sg1)
        ss = (ss0, ss1)

        def idx(i):
            return idx_all.at[pl.ds(i * _CHUNK, _CHUNK)]

        def gather_pair(i, b):
            return (pltpu.make_async_copy(mu_tab.at[idx(i)], mu_v.at[b], sg[b]),
                    pltpu.make_async_copy(var_tab.at[idx(i)], var_v.at[b], sg[b]))

        def scatter_pair(i, b):
            dst = pl.ds(base + i * _CHUNK, _CHUNK)
            return (pltpu.make_async_copy(mu_v.at[b], mu_out.at[dst], ss[b]),
                    pltpu.make_async_copy(var_v.at[b], var_out.at[dst], ss[b]))

        def start(pair):
            pair[0].start()
            pair[1].start()

        def wait(pair):
            pair[0].wait()
            pair[1].wait()

        start(gather_pair(0, 0))
        start(gather_pair(1, 1))
        wait(gather_pair(0, 0))
        start(scatter_pair(0, 0))

        def body(r, carry):
            for step in (1, 2):
                i = 2 * r + step
                b = step % 2
                wait(scatter_pair(i - 1, 1 - b))
                start(gather_pair(i + 1, 1 - b))
                wait(gather_pair(i, b))
                start(scatter_pair(i, b))
            return carry

        lax.fori_loop(0, (n_chunks - 2) // 2, body, 0)

        last = n_chunks - 1
        wait(gather_pair(last, last % 2))
        start(scatter_pair(last, last % 2))
        wait(scatter_pair(last - 1, (last - 1) % 2))
        wait(scatter_pair(last, last % 2))

    return gather_k


def kernel(ids, translation, log_var):
    var_table = pl.pallas_call(
        _var_table_body,
        out_shape=jax.ShapeDtypeStruct(log_var.shape, jnp.float32),
    )(log_var)

    info = plsc.get_sparse_core_info()
    num_rows = ids.size
    dim = translation.shape[1]
    ids_flat = ids.reshape(num_rows)
    gather_k = _make_gather(num_rows, dim, info.num_cores, info.num_subcores)
    mu_flat, var_flat = gather_k(ids_flat, translation, var_table)
    out_shape = ids.shape + (dim,)
    return mu_flat.reshape(out_shape), var_flat.reshape(out_shape)
